# baseline (speedup 1.0000x reference)
import jax
import jax.numpy as jnp
from jax.experimental import pallas as pl
from jax.experimental.pallas import tpu as pltpu

D = 512
N = 100000
K = 8
M = 16
BLK = 2000
NB = N // BLK
NCAND = 7
NEG = -3.0e38


def _sims_body(z_ref, mem_ref, idx_ref, sims):
    i = pl.program_id(0)
    blk = mem_ref[...]
    z = z_ref[...]
    dot = jax.lax.dot_general(
        z, blk, (((1,), (1,)), ((), ())),
        preferred_element_type=jnp.float32)
    ones = jnp.ones((1, D), jnp.float32)
    nsq = jax.lax.dot_general(
        ones, blk * blk, (((1,), (1,)), ((), ())),
        preferred_element_type=jnp.float32)
    sims[pl.ds(i, 1), :] = dot / (jnp.sqrt(nsq) + 1e-12)

    @pl.when(i == NB - 1)
    def _():
        s = sims[...]
        r = jax.lax.broadcasted_iota(jnp.int32, (NB, BLK), 0)
        c = jax.lax.broadcasted_iota(jnp.int32, (NB, BLK), 1)
        flat = r * BLK + c
        big = jnp.int32(2147483647)
        for k in range(M):
            v = jnp.max(s)
            fi = jnp.min(jnp.where(s == v, flat, big))
            idx_ref[k] = fi
            s = jnp.where(flat == fi, NEG, s)


def _tail_body(idx_ref, mem_grp, z, h, eps, Wq, bq, Wc, bc, Ws,
               Wm, bm, Wg1, bg1, Wg2r, bg2, s_out, alpha_out,
               protos, esims, ptop):
    i = pl.program_id(0)
    sub = idx_ref[i] % 8
    row = mem_grp[0, pl.ds(sub, 1), :]
    protos[pl.ds(i, 1), :] = row
    dot = jnp.sum(row * z[...])
    nsq = jnp.sum(row * row)
    esims[i] = dot / (jnp.sqrt(nsq) + 1e-12)

    @pl.when(i == M - 1)
    def _():
        big = jnp.int32(2147483647)
        for k in range(K):
            def sel(j, carry):
                bs_, bi_, bj_ = carry
                sj = esims[j]
                ij = idx_ref[j]
                better = jnp.logical_or(
                    sj > bs_, jnp.logical_and(sj == bs_, ij < bi_))
                return (jnp.where(better, sj, bs_),
                        jnp.where(better, ij, bi_),
                        jnp.where(better, j, bj_))
            _, _, bj = jax.lax.fori_loop(
                0, M, sel, (jnp.float32(NEG), big, jnp.int32(0)))
            esims[bj] = NEG
            ptop[pl.ds(k, 1), :] = protos[pl.ds(bj, 1), :]

        P = ptop[...]
        mean8 = jnp.mean(P, axis=0, keepdims=True)
        mean2 = jnp.mean(P[:2], axis=0, keepdims=True)
        mean3 = jnp.mean(P[:3], axis=0, keepdims=True)
        dreams = jnp.clip(z[...] + eps[...], -2.0, 2.0)
        C = jnp.concatenate(
            [mean8, mean2, mean3, dreams, jnp.zeros((1, D), jnp.float32)],
            axis=0)

        def mm(a, b):
            return jax.lax.dot_general(
                a, b, (((1,), (0,)), ((), ())),
                preferred_element_type=jnp.float32)

        qv = jnp.tanh(mm(z[...], Wq[:D, :]) + mm(h[...], Wq[D:, :])
                      + bq[...])
        A = jnp.tanh(mm(C, Wc[...]) + bc[...])
        w = qv * Ws[...]
        scores = jax.lax.dot_general(
            A, w, (((1,), (1,)), ((), ())),
            preferred_element_type=jnp.float32)
        rows = jax.lax.broadcasted_iota(jnp.int32, (K, 1), 0)
        scores = jnp.where(rows < NCAND, scores, NEG)
        m_ = jnp.max(scores)
        e = jnp.exp(scores - m_)
        alpha = e / jnp.sum(e)
        mix = jax.lax.dot_general(
            alpha, C, (((0,), (0,)), ((), ())),
            preferred_element_type=jnp.float32)
        raw = jnp.tanh(mm(mix, Wm[...]) + bm[...])
        g1 = jnp.tanh(mm(z[...], Wg1[:D, :]) + mm(h[...], Wg1[D:, :])
                      + bg1[...])
        gl = jnp.sum(g1 * Wg2r[...]) + bg2[0]
        gate = jax.nn.sigmoid(gl)
        s_out[...] = gate * raw
        alpha_out[...] = alpha


def kernel(z_t, h_t, mem_bank, Wq, bq, Wc, bc, Ws, bs, Wm, bm,
           Wg1, bg1, Wg2, bg2):
    z2 = z_t.reshape(1, D)
    h2 = h_t.reshape(1, D)

    idx = pl.pallas_call(
        _sims_body,
        grid=(NB,),
        in_specs=[
            pl.BlockSpec((1, D), lambda i: (0, 0)),
            pl.BlockSpec((BLK, D), lambda i: (i, 0)),
        ],
        out_specs=pl.BlockSpec(memory_space=pltpu.SMEM),
        out_shape=jax.ShapeDtypeStruct((M,), jnp.int32),
        scratch_shapes=[pltpu.VMEM((NB, BLK), jnp.float32)],
        compiler_params=pltpu.CompilerParams(
            dimension_semantics=("arbitrary",)),
    )(z2, mem_bank)

    eps = 0.08 * jax.random.normal(jax.random.key(1), (4, D), jnp.float32)

    grid_spec = pltpu.PrefetchScalarGridSpec(
        num_scalar_prefetch=1,
        grid=(M,),
        in_specs=[
            pl.BlockSpec((1, 8, D), lambda i, idx_ref: (idx_ref[i] // 8, 0, 0)),
            pl.BlockSpec((1, D), lambda i, idx_ref: (0, 0)),
            pl.BlockSpec((1, D), lambda i, idx_ref: (0, 0)),
            pl.BlockSpec((4, D), lambda i, idx_ref: (0, 0)),
            pl.BlockSpec((2 * D, D), lambda i, idx_ref: (0, 0)),
            pl.BlockSpec((1, D), lambda i, idx_ref: (0, 0)),
            pl.BlockSpec((D, D), lambda i, idx_ref: (0, 0)),
            pl.BlockSpec((1, D), lambda i, idx_ref: (0, 0)),
            pl.BlockSpec((1, D), lambda i, idx_ref: (0, 0)),
            pl.BlockSpec((D, D), lambda i, idx_ref: (0, 0)),
            pl.BlockSpec((1, D), lambda i, idx_ref: (0, 0)),
            pl.BlockSpec((2 * D, D), lambda i, idx_ref: (0, 0)),
            pl.BlockSpec((1, D), lambda i, idx_ref: (0, 0)),
            pl.BlockSpec((1, D), lambda i, idx_ref: (0, 0)),
            pl.BlockSpec(memory_space=pltpu.SMEM),
        ],
        out_specs=[
            pl.BlockSpec((1, D), lambda i, idx_ref: (0, 0)),
            pl.BlockSpec((K, 1), lambda i, idx_ref: (0, 0)),
        ],
        scratch_shapes=[
            pltpu.VMEM((M, D), jnp.float32),
            pltpu.SMEM((M,), jnp.float32),
            pltpu.VMEM((K, D), jnp.float32),
        ],
    )

    s2, alpha8 = pl.pallas_call(
        _tail_body,
        grid_spec=grid_spec,
        out_shape=[
            jax.ShapeDtypeStruct((1, D), jnp.float32),
            jax.ShapeDtypeStruct((K, 1), jnp.float32),
        ],
    )(idx, mem_bank.reshape(N // 8, 8, D), z2, h2, eps,
      Wq, bq.reshape(1, D), Wc, bc.reshape(1, D), Ws.reshape(1, D),
      Wm, bm.reshape(1, D), Wg1, bg1.reshape(1, D),
      Wg2.reshape(1, D), bg2.reshape(1))

    return (s2.reshape(D), alpha8[:NCAND, 0])


# speedup vs baseline: 1.2513x; 1.2513x over previous
import jax
import jax.numpy as jnp
from jax.experimental import pallas as pl
from jax.experimental.pallas import tpu as pltpu

D = 512
N = 100000
K = 8
M = 16
BLK = 5000
NB = N // BLK
NCAND = 7
NEG = -3.0e38


def _sims_body(z_ref, mem_ref, idx_ref, sims):
    i = pl.program_id(0)
    blk = mem_ref[...]
    z = z_ref[...]
    dot = jax.lax.dot_general(
        z, blk, (((1,), (1,)), ((), ())),
        preferred_element_type=jnp.float32)
    ones = jnp.ones((1, D), jnp.float32)
    nsq = jax.lax.dot_general(
        ones, blk * blk, (((1,), (1,)), ((), ())),
        preferred_element_type=jnp.float32)
    sims[pl.ds(i, 1), :] = dot / (jnp.sqrt(nsq) + 1e-12)

    @pl.when(i == NB - 1)
    def _():
        s = sims[...]
        r = jax.lax.broadcasted_iota(jnp.int32, (NB, BLK), 0)
        c = jax.lax.broadcasted_iota(jnp.int32, (NB, BLK), 1)
        flat = r * BLK + c
        big = jnp.int32(2147483647)
        for k in range(M):
            v = jnp.max(s)
            fi = jnp.min(jnp.where(s == v, flat, big))
            idx_ref[k] = fi
            s = jnp.where(flat == fi, NEG, s)


def _tail_body(idx_ref, *refs):
    groups = refs[:M]
    (z, h, eps, Wq, bq, Wc, bc, Ws, Wm, bm, Wg1, bg1, Wg2r, bg2,
     s_out, alpha_out, protos, esims, ptop) = refs[M:]

    zv = z[...]
    for j in range(M):
        sub = idx_ref[j] % 8
        row = groups[j][0, pl.ds(sub, 1), :]
        protos[pl.ds(j, 1), :] = row
        dot = jnp.sum(row * zv)
        nsq = jnp.sum(row * row)
        esims[j] = dot / (jnp.sqrt(nsq) + 1e-12)

    big = jnp.int32(2147483647)
    for k in range(K):
        def sel(j, carry):
            bs_, bi_, bj_ = carry
            sj = esims[j]
            ij = idx_ref[j]
            better = jnp.logical_or(
                sj > bs_, jnp.logical_and(sj == bs_, ij < bi_))
            return (jnp.where(better, sj, bs_),
                    jnp.where(better, ij, bi_),
                    jnp.where(better, j, bj_))
        _, _, bj = jax.lax.fori_loop(
            0, M, sel, (jnp.float32(NEG), big, jnp.int32(0)))
        esims[bj] = NEG
        ptop[pl.ds(k, 1), :] = protos[pl.ds(bj, 1), :]

    P = ptop[...]
    mean8 = jnp.mean(P, axis=0, keepdims=True)
    mean2 = jnp.mean(P[:2], axis=0, keepdims=True)
    mean3 = jnp.mean(P[:3], axis=0, keepdims=True)
    dreams = jnp.clip(zv + eps[...], -2.0, 2.0)
    C = jnp.concatenate(
        [mean8, mean2, mean3, dreams, jnp.zeros((1, D), jnp.float32)],
        axis=0)

    def mm(a, b):
        return jax.lax.dot_general(
            a, b, (((1,), (0,)), ((), ())),
            preferred_element_type=jnp.float32)

    qv = jnp.tanh(mm(zv, Wq[:D, :]) + mm(h[...], Wq[D:, :])
                  + bq[...])
    A = jnp.tanh(mm(C, Wc[...]) + bc[...])
    w = qv * Ws[...]
    scores = jax.lax.dot_general(
        A, w, (((1,), (1,)), ((), ())),
        preferred_element_type=jnp.float32)
    rows = jax.lax.broadcasted_iota(jnp.int32, (K, 1), 0)
    scores = jnp.where(rows < NCAND, scores, NEG)
    m_ = jnp.max(scores)
    e = jnp.exp(scores - m_)
    alpha = e / jnp.sum(e)
    mix = jax.lax.dot_general(
        alpha, C, (((0,), (0,)), ((), ())),
        preferred_element_type=jnp.float32)
    raw = jnp.tanh(mm(mix, Wm[...]) + bm[...])
    g1 = jnp.tanh(mm(zv, Wg1[:D, :]) + mm(h[...], Wg1[D:, :])
                  + bg1[...])
    gl = jnp.sum(g1 * Wg2r[...]) + bg2[0]
    gate = jax.nn.sigmoid(gl)
    s_out[...] = gate * raw
    alpha_out[...] = alpha


def _group_spec(j):
    return pl.BlockSpec(
        (1, 8, D), lambda i, idx_ref, j=j: (idx_ref[j] // 8, 0, 0))


def kernel(z_t, h_t, mem_bank, Wq, bq, Wc, bc, Ws, bs, Wm, bm,
           Wg1, bg1, Wg2, bg2):
    z2 = z_t.reshape(1, D)
    h2 = h_t.reshape(1, D)

    idx = pl.pallas_call(
        _sims_body,
        grid=(NB,),
        in_specs=[
            pl.BlockSpec((1, D), lambda i: (0, 0)),
            pl.BlockSpec((BLK, D), lambda i: (i, 0)),
        ],
        out_specs=pl.BlockSpec(memory_space=pltpu.SMEM),
        out_shape=jax.ShapeDtypeStruct((M,), jnp.int32),
        scratch_shapes=[pltpu.VMEM((NB, BLK), jnp.float32)],
        compiler_params=pltpu.CompilerParams(
            dimension_semantics=("arbitrary",)),
    )(z2, mem_bank)

    eps = 0.08 * jax.random.normal(jax.random.key(1), (4, D), jnp.float32)

    grid_spec = pltpu.PrefetchScalarGridSpec(
        num_scalar_prefetch=1,
        grid=(1,),
        in_specs=[_group_spec(j) for j in range(M)] + [
            pl.BlockSpec((1, D), lambda i, idx_ref: (0, 0)),
            pl.BlockSpec((1, D), lambda i, idx_ref: (0, 0)),
            pl.BlockSpec((4, D), lambda i, idx_ref: (0, 0)),
            pl.BlockSpec((2 * D, D), lambda i, idx_ref: (0, 0)),
            pl.BlockSpec((1, D), lambda i, idx_ref: (0, 0)),
            pl.BlockSpec((D, D), lambda i, idx_ref: (0, 0)),
            pl.BlockSpec((1, D), lambda i, idx_ref: (0, 0)),
            pl.BlockSpec((1, D), lambda i, idx_ref: (0, 0)),
            pl.BlockSpec((D, D), lambda i, idx_ref: (0, 0)),
            pl.BlockSpec((1, D), lambda i, idx_ref: (0, 0)),
            pl.BlockSpec((2 * D, D), lambda i, idx_ref: (0, 0)),
            pl.BlockSpec((1, D), lambda i, idx_ref: (0, 0)),
            pl.BlockSpec((1, D), lambda i, idx_ref: (0, 0)),
            pl.BlockSpec(memory_space=pltpu.SMEM),
        ],
        out_specs=[
            pl.BlockSpec((1, D), lambda i, idx_ref: (0, 0)),
            pl.BlockSpec((K, 1), lambda i, idx_ref: (0, 0)),
        ],
        scratch_shapes=[
            pltpu.VMEM((M, D), jnp.float32),
            pltpu.SMEM((M,), jnp.float32),
            pltpu.VMEM((K, D), jnp.float32),
        ],
    )

    mem3 = mem_bank.reshape(N // 8, 8, D)
    s2, alpha8 = pl.pallas_call(
        _tail_body,
        grid_spec=grid_spec,
        out_shape=[
            jax.ShapeDtypeStruct((1, D), jnp.float32),
            jax.ShapeDtypeStruct((K, 1), jnp.float32),
        ],
    )(idx, *([mem3] * M), z2, h2, eps,
      Wq, bq.reshape(1, D), Wc, bc.reshape(1, D), Ws.reshape(1, D),
      Wm, bm.reshape(1, D), Wg1, bg1.reshape(1, D),
      Wg2.reshape(1, D), bg2.reshape(1))

    return (s2.reshape(D), alpha8[:NCAND, 0])


# speedup vs baseline: 1.2819x; 1.0245x over previous
import jax
import jax.numpy as jnp
from jax.experimental import pallas as pl
from jax.experimental.pallas import tpu as pltpu

D = 512
N = 100000
K = 8
M = 16
BLK = 10000
NB = N // BLK
NCAND = 7
NEG = -3.0e38


def _sims_body(z_ref, mem_ref, idx_ref, sims):
    i = pl.program_id(0)
    blk = mem_ref[...]
    z = z_ref[...]
    dot = jax.lax.dot_general(
        z, blk, (((1,), (1,)), ((), ())),
        preferred_element_type=jnp.float32)
    ones = jnp.ones((1, D), jnp.float32)
    nsq = jax.lax.dot_general(
        ones, blk * blk, (((1,), (1,)), ((), ())),
        preferred_element_type=jnp.float32)
    sims[pl.ds(i, 1), :] = dot / (jnp.sqrt(nsq) + 1e-12)

    @pl.when(i == NB - 1)
    def _():
        s = sims[...]
        r = jax.lax.broadcasted_iota(jnp.int32, (NB, BLK), 0)
        c = jax.lax.broadcasted_iota(jnp.int32, (NB, BLK), 1)
        flat = r * BLK + c
        big = jnp.int32(2147483647)
        for k in range(M):
            v = jnp.max(s)
            fi = jnp.min(jnp.where(s == v, flat, big))
            idx_ref[k] = fi
            s = jnp.where(flat == fi, NEG, s)


def _tail_body(idx_ref, *refs):
    groups = refs[:M]
    (z, h, eps, Wq, bq, Wc, bc, Ws, Wm, bm, Wg1, bg1, Wg2r, bg2,
     s_out, alpha_out, protos, esims, ptop) = refs[M:]

    zv = z[...]
    for j in range(M):
        sub = idx_ref[j] % 8
        row = groups[j][0, pl.ds(sub, 1), :]
        protos[pl.ds(j, 1), :] = row
        dot = jnp.sum(row * zv)
        nsq = jnp.sum(row * row)
        esims[j] = dot / (jnp.sqrt(nsq) + 1e-12)

    big = jnp.int32(2147483647)
    for k in range(K):
        def sel(j, carry):
            bs_, bi_, bj_ = carry
            sj = esims[j]
            ij = idx_ref[j]
            better = jnp.logical_or(
                sj > bs_, jnp.logical_and(sj == bs_, ij < bi_))
            return (jnp.where(better, sj, bs_),
                    jnp.where(better, ij, bi_),
                    jnp.where(better, j, bj_))
        _, _, bj = jax.lax.fori_loop(
            0, M, sel, (jnp.float32(NEG), big, jnp.int32(0)))
        esims[bj] = NEG
        ptop[pl.ds(k, 1), :] = protos[pl.ds(bj, 1), :]

    P = ptop[...]
    mean8 = jnp.mean(P, axis=0, keepdims=True)
    mean2 = jnp.mean(P[:2], axis=0, keepdims=True)
    mean3 = jnp.mean(P[:3], axis=0, keepdims=True)
    dreams = jnp.clip(zv + eps[...], -2.0, 2.0)
    C = jnp.concatenate(
        [mean8, mean2, mean3, dreams, jnp.zeros((1, D), jnp.float32)],
        axis=0)

    def mm(a, b):
        return jax.lax.dot_general(
            a, b, (((1,), (0,)), ((), ())),
            preferred_element_type=jnp.float32)

    qv = jnp.tanh(mm(zv, Wq[:D, :]) + mm(h[...], Wq[D:, :])
                  + bq[...])
    A = jnp.tanh(mm(C, Wc[...]) + bc[...])
    w = qv * Ws[...]
    scores = jax.lax.dot_general(
        A, w, (((1,), (1,)), ((), ())),
        preferred_element_type=jnp.float32)
    rows = jax.lax.broadcasted_iota(jnp.int32, (K, 1), 0)
    scores = jnp.where(rows < NCAND, scores, NEG)
    m_ = jnp.max(scores)
    e = jnp.exp(scores - m_)
    alpha = e / jnp.sum(e)
    mix = jax.lax.dot_general(
        alpha, C, (((0,), (0,)), ((), ())),
        preferred_element_type=jnp.float32)
    raw = jnp.tanh(mm(mix, Wm[...]) + bm[...])
    g1 = jnp.tanh(mm(zv, Wg1[:D, :]) + mm(h[...], Wg1[D:, :])
                  + bg1[...])
    gl = jnp.sum(g1 * Wg2r[...]) + bg2[0]
    gate = jax.nn.sigmoid(gl)
    s_out[...] = gate * raw
    alpha_out[...] = alpha


def _group_spec(j):
    return pl.BlockSpec(
        (1, 8, D), lambda i, idx_ref, j=j: (idx_ref[j] // 8, 0, 0))


def kernel(z_t, h_t, mem_bank, Wq, bq, Wc, bc, Ws, bs, Wm, bm,
           Wg1, bg1, Wg2, bg2):
    z2 = z_t.reshape(1, D)
    h2 = h_t.reshape(1, D)

    idx = pl.pallas_call(
        _sims_body,
        grid=(NB,),
        in_specs=[
            pl.BlockSpec((1, D), lambda i: (0, 0)),
            pl.BlockSpec((BLK, D), lambda i: (i, 0)),
        ],
        out_specs=pl.BlockSpec(memory_space=pltpu.SMEM),
        out_shape=jax.ShapeDtypeStruct((M,), jnp.int32),
        scratch_shapes=[pltpu.VMEM((NB, BLK), jnp.float32)],
        compiler_params=pltpu.CompilerParams(
            dimension_semantics=("arbitrary",)),
    )(z2, mem_bank)

    eps = 0.08 * jax.random.normal(jax.random.key(1), (4, D), jnp.float32)

    grid_spec = pltpu.PrefetchScalarGridSpec(
        num_scalar_prefetch=1,
        grid=(1,),
        in_specs=[_group_spec(j) for j in range(M)] + [
            pl.BlockSpec((1, D), lambda i, idx_ref: (0, 0)),
            pl.BlockSpec((1, D), lambda i, idx_ref: (0, 0)),
            pl.BlockSpec((4, D), lambda i, idx_ref: (0, 0)),
            pl.BlockSpec((2 * D, D), lambda i, idx_ref: (0, 0)),
            pl.BlockSpec((1, D), lambda i, idx_ref: (0, 0)),
            pl.BlockSpec((D, D), lambda i, idx_ref: (0, 0)),
            pl.BlockSpec((1, D), lambda i, idx_ref: (0, 0)),
            pl.BlockSpec((1, D), lambda i, idx_ref: (0, 0)),
            pl.BlockSpec((D, D), lambda i, idx_ref: (0, 0)),
            pl.BlockSpec((1, D), lambda i, idx_ref: (0, 0)),
            pl.BlockSpec((2 * D, D), lambda i, idx_ref: (0, 0)),
            pl.BlockSpec((1, D), lambda i, idx_ref: (0, 0)),
            pl.BlockSpec((1, D), lambda i, idx_ref: (0, 0)),
            pl.BlockSpec(memory_space=pltpu.SMEM),
        ],
        out_specs=[
            pl.BlockSpec((1, D), lambda i, idx_ref: (0, 0)),
            pl.BlockSpec((K, 1), lambda i, idx_ref: (0, 0)),
        ],
        scratch_shapes=[
            pltpu.VMEM((M, D), jnp.float32),
            pltpu.SMEM((M,), jnp.float32),
            pltpu.VMEM((K, D), jnp.float32),
        ],
    )

    mem3 = mem_bank.reshape(N // 8, 8, D)
    s2, alpha8 = pl.pallas_call(
        _tail_body,
        grid_spec=grid_spec,
        out_shape=[
            jax.ShapeDtypeStruct((1, D), jnp.float32),
            jax.ShapeDtypeStruct((K, 1), jnp.float32),
        ],
    )(idx, *([mem3] * M), z2, h2, eps,
      Wq, bq.reshape(1, D), Wc, bc.reshape(1, D), Ws.reshape(1, D),
      Wm, bm.reshape(1, D), Wg1, bg1.reshape(1, D),
      Wg2.reshape(1, D), bg2.reshape(1))

    return (s2.reshape(D), alpha8[:NCAND, 0])


# speedup vs baseline: 1.3086x; 1.0208x over previous
import jax
import jax.numpy as jnp
from jax.experimental import pallas as pl
from jax.experimental.pallas import tpu as pltpu

D = 512
N = 100000
K = 8
M = 16
BLK = 10000
NB = N // BLK
NCAND = 7
NEG = -3.0e38


def _body(z_ref, mem_ref, h, eps, Wq, bq, Wc, bc, Ws, Wm, bm,
          Wg1, bg1, Wg2r, bg2, mem_any, s_out, alpha_out,
          sims, protos, esims, idx_smem, ptop, sem):
    i = pl.program_id(0)
    blk = mem_ref[...]
    zv = z_ref[...]
    dot = jax.lax.dot_general(
        zv, blk, (((1,), (1,)), ((), ())),
        preferred_element_type=jnp.float32)
    ones = jnp.ones((1, D), jnp.float32)
    nsq = jax.lax.dot_general(
        ones, blk * blk, (((1,), (1,)), ((), ())),
        preferred_element_type=jnp.float32)
    sims[pl.ds(i, 1), :] = dot / (jnp.sqrt(nsq) + 1e-12)

    @pl.when(i == NB - 1)
    def _():
        s = sims[...]
        r = jax.lax.broadcasted_iota(jnp.int32, (NB, BLK), 0)
        c = jax.lax.broadcasted_iota(jnp.int32, (NB, BLK), 1)
        flat = r * BLK + c
        big = jnp.int32(2147483647)
        for k in range(M):
            v = jnp.max(s)
            fi = jnp.min(jnp.where(s == v, flat, big))
            idx_smem[k] = fi
            s = jnp.where(flat == fi, NEG, s)

        for j in range(M):
            pltpu.make_async_copy(
                mem_any.at[pl.ds(idx_smem[j], 1), :],
                protos.at[pl.ds(j, 1), :], sem).start()
        for j in range(M):
            pltpu.make_async_copy(
                mem_any.at[pl.ds(idx_smem[j], 1), :],
                protos.at[pl.ds(j, 1), :], sem).wait()

        for j in range(M):
            row = protos[pl.ds(j, 1), :]
            d_ = jnp.sum(row * zv)
            n_ = jnp.sum(row * row)
            esims[j] = d_ / (jnp.sqrt(n_) + 1e-12)

        for k in range(K):
            def sel(j, carry):
                bs_, bi_, bj_ = carry
                sj = esims[j]
                ij = idx_smem[j]
                better = jnp.logical_or(
                    sj > bs_, jnp.logical_and(sj == bs_, ij < bi_))
                return (jnp.where(better, sj, bs_),
                        jnp.where(better, ij, bi_),
                        jnp.where(better, j, bj_))
            _, _, bj = jax.lax.fori_loop(
                0, M, sel, (jnp.float32(NEG), big, jnp.int32(0)))
            esims[bj] = NEG
            ptop[pl.ds(k, 1), :] = protos[pl.ds(bj, 1), :]

        P = ptop[...]
        mean8 = jnp.mean(P, axis=0, keepdims=True)
        mean2 = jnp.mean(P[:2], axis=0, keepdims=True)
        mean3 = jnp.mean(P[:3], axis=0, keepdims=True)
        dreams = jnp.clip(zv + eps[...], -2.0, 2.0)
        C = jnp.concatenate(
            [mean8, mean2, mean3, dreams, jnp.zeros((1, D), jnp.float32)],
            axis=0)

        def mm(a, b):
            return jax.lax.dot_general(
                a, b, (((1,), (0,)), ((), ())),
                preferred_element_type=jnp.float32)

        qv = jnp.tanh(mm(zv, Wq[:D, :]) + mm(h[...], Wq[D:, :])
                      + bq[...])
        A = jnp.tanh(mm(C, Wc[...]) + bc[...])
        w = qv * Ws[...]
        scores = jax.lax.dot_general(
            A, w, (((1,), (1,)), ((), ())),
            preferred_element_type=jnp.float32)
        rows = jax.lax.broadcasted_iota(jnp.int32, (K, 1), 0)
        scores = jnp.where(rows < NCAND, scores, NEG)
        m_ = jnp.max(scores)
        e = jnp.exp(scores - m_)
        alpha = e / jnp.sum(e)
        mix = jax.lax.dot_general(
            alpha, C, (((0,), (0,)), ((), ())),
            preferred_element_type=jnp.float32)
        raw = jnp.tanh(mm(mix, Wm[...]) + bm[...])
        g1 = jnp.tanh(mm(zv, Wg1[:D, :]) + mm(h[...], Wg1[D:, :])
                      + bg1[...])
        gl = jnp.sum(g1 * Wg2r[...]) + bg2[0]
        gate = jax.nn.sigmoid(gl)
        s_out[...] = gate * raw
        alpha_out[...] = alpha


def kernel(z_t, h_t, mem_bank, Wq, bq, Wc, bc, Ws, bs, Wm, bm,
           Wg1, bg1, Wg2, bg2):
    z2 = z_t.reshape(1, D)
    h2 = h_t.reshape(1, D)
    eps = 0.08 * jax.random.normal(jax.random.key(1), (4, D), jnp.float32)

    s2, alpha8 = pl.pallas_call(
        _body,
        grid=(NB,),
        in_specs=[
            pl.BlockSpec((1, D), lambda i: (0, 0)),
            pl.BlockSpec((BLK, D), lambda i: (i, 0)),
            pl.BlockSpec((1, D), lambda i: (0, 0)),
            pl.BlockSpec((4, D), lambda i: (0, 0)),
            pl.BlockSpec((2 * D, D), lambda i: (0, 0)),
            pl.BlockSpec((1, D), lambda i: (0, 0)),
            pl.BlockSpec((D, D), lambda i: (0, 0)),
            pl.BlockSpec((1, D), lambda i: (0, 0)),
            pl.BlockSpec((1, D), lambda i: (0, 0)),
            pl.BlockSpec((D, D), lambda i: (0, 0)),
            pl.BlockSpec((1, D), lambda i: (0, 0)),
            pl.BlockSpec((2 * D, D), lambda i: (0, 0)),
            pl.BlockSpec((1, D), lambda i: (0, 0)),
            pl.BlockSpec((1, D), lambda i: (0, 0)),
            pl.BlockSpec(memory_space=pltpu.SMEM),
            pl.BlockSpec(memory_space=pltpu.MemorySpace.HBM),
        ],
        out_specs=[
            pl.BlockSpec((1, D), lambda i: (0, 0)),
            pl.BlockSpec((K, 1), lambda i: (0, 0)),
        ],
        out_shape=[
            jax.ShapeDtypeStruct((1, D), jnp.float32),
            jax.ShapeDtypeStruct((K, 1), jnp.float32),
        ],
        scratch_shapes=[
            pltpu.VMEM((NB, BLK), jnp.float32),
            pltpu.VMEM((M, D), jnp.float32),
            pltpu.SMEM((M,), jnp.float32),
            pltpu.SMEM((M,), jnp.int32),
            pltpu.VMEM((K, D), jnp.float32),
            pltpu.SemaphoreType.DMA,
        ],
        compiler_params=pltpu.CompilerParams(
            dimension_semantics=("arbitrary",)),
    )(z2, mem_bank, h2, eps,
      Wq, bq.reshape(1, D), Wc, bc.reshape(1, D), Ws.reshape(1, D),
      Wm, bm.reshape(1, D), Wg1, bg1.reshape(1, D),
      Wg2.reshape(1, D), bg2.reshape(1), mem_bank)

    return (s2.reshape(D), alpha8[:NCAND, 0])
